# baseline (speedup 1.0000x reference)
import dataclasses
import functools

import jax
import jax.numpy as jnp
from jax import lax
from jax.experimental import pallas as pl
from jax.experimental.pallas import tpu as pltpu
from jax.experimental.pallas import tpu_sc as plsc

N = 10000
E = 320000
G = 64
CIN = 128
H = 128
HH = 64
OUT = 32
NC, NS = 2, 16
NW = NC * NS
CH = 128
NCHUNK = 2560
EPAD = NCHUNK * CH
C16 = NCHUNK // 16
C32 = NCHUNK // 32
NPAD = 10240
RPT = NPAD // NS
VSIZE = G * NPAD
VPT = VSIZE // NS
NB = 1024

_mesh = plsc.VectorSubcoreMesh(core_axis_name="core", subcore_axis_name="subcore")

_sc_params = pltpu.CompilerParams()
if "needs_layout_passes" in pltpu.CompilerParams.__dataclass_fields__:
    _sc_params = dataclasses.replace(_sc_params, needs_layout_passes=False)


@functools.partial(
    pl.kernel,
    out_type=jax.ShapeDtypeStruct((NC, NPAD), jnp.float32),
    mesh=_mesh,
    scratch_types=[
        pltpu.VMEM((C32, CH), jnp.int32),
        pltpu.VMEM((CH,), jnp.float32),
        pltpu.VMEM_SHARED((NPAD,), jnp.float32),
        pltpu.SemaphoreType.DMA,
    ],
    compiler_params=_sc_params,
)
def _deg_call(dst2_hbm, ones_hbm, z_hbm, deg_out, dstx, onesv, deg_sh, sem):
    c = lax.axis_index("core")
    s = lax.axis_index("subcore")
    w = c * NS + s
    off = w * C32
    pltpu.sync_copy(z_hbm, deg_sh.at[pl.ds(s * RPT, RPT)])
    pltpu.sync_copy(ones_hbm, onesv)
    pltpu.sync_copy(dst2_hbm.at[pl.ds(off, C32)], dstx)
    plsc.subcore_barrier()

    @pl.loop(0, C32)
    def _(j):
        pltpu.async_copy(onesv, deg_sh.at[dstx.at[j]], sem, add=True)

    @pl.loop(0, C32)
    def _(j):
        pltpu.make_async_copy(onesv, deg_sh.at[dstx.at[0]], sem).wait()

    plsc.subcore_barrier()
    pltpu.sync_copy(deg_sh.at[pl.ds(s * RPT, RPT)],
                    deg_out.at[c, pl.ds(s * RPT, RPT)])


def _mm_body(x_ref, w_ref, h_ref):
    h_ref[...] = jnp.dot(x_ref[...], w_ref[...],
                         preferred_element_type=jnp.float32)


_mm_call = pl.pallas_call(
    _mm_body,
    grid=(NPAD // NB,),
    in_specs=[
        pl.BlockSpec((NB, CIN), lambda i: (i, 0)),
        pl.BlockSpec((CIN, H), lambda i: (0, 0)),
    ],
    out_specs=pl.BlockSpec((NB, H), lambda i: (i, 0)),
    out_shape=jax.ShapeDtypeStruct((NPAD, H), jnp.float32),
)


def _scale_body(h_ref, d0_ref, d1_ref, h2_ref, dinv_ref):
    dinv = lax.rsqrt(d0_ref[...] + d1_ref[...] + 1.0)
    h2_ref[...] = dinv * h_ref[...]
    dinv_ref[...] = dinv


_scale_call = pl.pallas_call(
    _scale_body,
    grid=(NPAD // NB,),
    in_specs=[
        pl.BlockSpec((NB, H), lambda i: (i, 0)),
        pl.BlockSpec((NB, 1), lambda i: (i, 0)),
        pl.BlockSpec((NB, 1), lambda i: (i, 0)),
    ],
    out_specs=[
        pl.BlockSpec((NB, H), lambda i: (i, 0)),
        pl.BlockSpec((NB, 1), lambda i: (i, 0)),
    ],
    out_shape=[
        jax.ShapeDtypeStruct((NPAD, H), jnp.float32),
        jax.ShapeDtypeStruct((NPAD, 1), jnp.float32),
    ],
)


@functools.partial(
    pl.kernel,
    out_type=jax.ShapeDtypeStruct((NC, NPAD, H), jnp.float32),
    mesh=_mesh,
    scratch_types=[
        pltpu.VMEM((4, CH), jnp.int32),
        pltpu.VMEM((4, CH), jnp.int32),
        pltpu.VMEM((CH, H), jnp.float32),
        pltpu.VMEM((CH, H), jnp.float32),
        pltpu.VMEM_SHARED((NPAD, H), jnp.float32),
        pltpu.SemaphoreType.DMA,
        pltpu.SemaphoreType.DMA,
        pltpu.SemaphoreType.DMA,
        pltpu.SemaphoreType.DMA,
        pltpu.SemaphoreType.DMA,
        pltpu.SemaphoreType.DMA,
        pltpu.SemaphoreType.DMA,
        pltpu.SemaphoreType.DMA,
    ],
    compiler_params=_sc_params,
)
def _agg_call(src1_hbm, dst1_hbm, h2_hbm, z2_hbm, agg_out,
              srcx, dstx, r0, r1, agg_sh,
              si0, si1, si2, si3, sg0, sg1, sc0, sc1):
    c = lax.axis_index("core")
    s = lax.axis_index("subcore")
    w = c * NS + s
    base = w * C32 * CH
    rows = (r0, r1)
    si = (si0, si1, si2, si3)
    sg = (sg0, sg1)
    sc = (sc0, sc1)

    def fire_idx(j, slot):
        e = pl.multiple_of(base + j * CH, CH)
        pltpu.async_copy(src1_hbm.at[pl.ds(e, CH)], srcx.at[slot], si[slot])
        pltpu.async_copy(dst1_hbm.at[pl.ds(e, CH)], dstx.at[slot], si[slot])

    def wait_idx(slot):
        pltpu.make_async_copy(src1_hbm.at[pl.ds(0, CH)], srcx.at[slot],
                              si[slot]).wait()
        pltpu.make_async_copy(dst1_hbm.at[pl.ds(0, CH)], dstx.at[slot],
                              si[slot]).wait()

    pltpu.sync_copy(z2_hbm, agg_sh.at[pl.ds(s * RPT, RPT)])
    plsc.subcore_barrier()

    for slot in range(4):
        fire_idx(slot, slot)

    @pl.loop(0, C32 // 4)
    def _(i):
        for b in range(4):
            j = 4 * i + b
            rb = b % 2
            done = pl.when(i >= 1) if b < 2 else (lambda f: f())

            @done
            def _():
                pltpu.make_async_copy(rows[rb], agg_sh.at[dstx.at[0]],
                                      sc[rb]).wait()
                if b < 2:
                    fire_idx(j + 2, (b + 2) % 4)

            if b >= 2:
                @pl.when(i < C32 // 4 - 1)
                def _():
                    fire_idx(j + 2, (b + 2) % 4)

            wait_idx(b)
            pltpu.async_copy(h2_hbm.at[srcx.at[b]], rows[rb], sg[rb])
            pltpu.make_async_copy(h2_hbm.at[srcx.at[0]], rows[rb],
                                  sg[rb]).wait()
            pltpu.async_copy(rows[rb], agg_sh.at[dstx.at[b]], sc[rb],
                             add=True)

    pltpu.make_async_copy(r0, agg_sh.at[dstx.at[0]], sc0).wait()
    pltpu.make_async_copy(r1, agg_sh.at[dstx.at[0]], sc1).wait()
    plsc.subcore_barrier()
    pltpu.sync_copy(agg_sh.at[pl.ds(s * RPT, RPT)],
                    agg_out.at[c, pl.ds(s * RPT, RPT)])


@functools.partial(
    pl.kernel,
    out_type=jax.ShapeDtypeStruct((NC, VSIZE), jnp.float32),
    mesh=_mesh,
    scratch_types=[
        pltpu.VMEM((C32, CH), jnp.int32),
        pltpu.VMEM((C32, CH), jnp.int32),
        pltpu.VMEM((C32, CH), jnp.float32),
        pltpu.VMEM((C32, CH), jnp.int32),
        pltpu.VMEM((NPAD,), jnp.float32),
        pltpu.VMEM((NPAD,), jnp.int32),
        pltpu.VMEM_SHARED((VSIZE,), jnp.float32),
        pltpu.SemaphoreType.DMA,
    ],
    compiler_params=_sc_params,
)
def _vtab_call(src2_hbm, dst2_hbm, dinv_hbm, batch_hbm, z1_hbm, v_out,
               srcx, dstx, vvals, vidx, dinvv, batchv, v_sh, sem):
    c = lax.axis_index("core")
    s = lax.axis_index("subcore")
    w = c * NS + s
    off = w * C32
    pltpu.sync_copy(z1_hbm, v_sh.at[pl.ds(s * VPT, VPT)])
    pltpu.sync_copy(dinv_hbm, dinvv)
    pltpu.sync_copy(batch_hbm, batchv)
    pltpu.sync_copy(src2_hbm.at[pl.ds(off, C32)], srcx)
    pltpu.sync_copy(dst2_hbm.at[pl.ds(off, C32)], dstx)

    zf16 = jnp.zeros((16,), jnp.float32)
    zi16 = jnp.zeros((16,), jnp.int32)
    for k in range((NPAD - N) // 16):
        dinvv[pl.ds(N + 16 * k, 16)] = zf16
        batchv[pl.ds(N + 16 * k, 16)] = zi16
    plsc.subcore_barrier()

    @pl.loop(0, C32)
    def _(j):
        @pl.loop(0, CH // 16)
        def _(k):
            s16 = srcx[j, pl.ds(16 * k, 16)]
            d16 = dstx[j, pl.ds(16 * k, 16)]
            vvals[j, pl.ds(16 * k, 16)] = plsc.load_gather(dinvv, [d16])
            vidx[j, pl.ds(16 * k, 16)] = (
                plsc.load_gather(batchv, [d16]) * NPAD + s16)

    @pl.loop(0, C32)
    def _(j):
        pltpu.async_copy(vvals.at[j], v_sh.at[vidx.at[j]], sem, add=True)

    @pl.loop(0, C32)
    def _(j):
        pltpu.make_async_copy(vvals.at[0], v_sh.at[vidx.at[0]], sem).wait()

    plsc.subcore_barrier()
    pltpu.sync_copy(v_sh.at[pl.ds(s * VPT, VPT)],
                    v_out.at[c, pl.ds(s * VPT, VPT)])


def _final_body(a0_ref, a1_ref, h2_ref, dcol_ref, drow_ref, brow_ref, v0_ref,
                v1_ref, b1_ref, w2g_ref, w2t_ref, b2g_ref, b2t_ref, out_ref,
                u_acc, cnt_acc):
    i = pl.program_id(0)

    @pl.when(i == 0)
    def _():
        u_acc[...] = jnp.zeros_like(u_acc)
        cnt_acc[...] = jnp.zeros_like(cnt_acc)

    dinv = dcol_ref[...]
    a = a0_ref[...] + a1_ref[...] + h2_ref[...]
    rd = dinv * jnp.maximum(dinv * a + b1_ref[...], 0.0)
    gids = lax.broadcasted_iota(jnp.int32, (G, NB), 0)
    onehot = brow_ref[...] == gids
    v_eff = v0_ref[...] + v1_ref[...] + jnp.where(onehot, drow_ref[...], 0.0)
    u_acc[...] += jnp.dot(v_eff, rd, preferred_element_type=jnp.float32)
    cnt_acc[...] += jnp.sum(onehot.astype(jnp.float32), axis=1, keepdims=True)

    @pl.when(i == pl.num_programs(0) - 1)
    def _():
        cnt = cnt_acc[...]
        cinv = 1.0 / jnp.maximum(cnt, 1.0)
        nz = jnp.where(cnt > 0, 1.0, 0.0)
        us = u_acc[...] * cinv
        pg = jnp.dot(us[:, :HH], w2g_ref[...],
                     preferred_element_type=jnp.float32) + b2g_ref[...] * nz
        pt = jnp.dot(us[:, HH:], w2t_ref[...],
                     preferred_element_type=jnp.float32) + b2t_ref[...] * nz
        diff = pt - pg + 1e-6
        dist = jnp.sqrt(jnp.sum(diff * diff, axis=1, keepdims=True))
        out_ref[...] = jnp.sum(dist).reshape(1, 1) / G


_final_call = pl.pallas_call(
    _final_body,
    grid=(NPAD // NB,),
    in_specs=[
        pl.BlockSpec((NB, H), lambda i: (i, 0)),
        pl.BlockSpec((NB, H), lambda i: (i, 0)),
        pl.BlockSpec((NB, H), lambda i: (i, 0)),
        pl.BlockSpec((NB, 1), lambda i: (i, 0)),
        pl.BlockSpec((1, NB), lambda i: (0, i)),
        pl.BlockSpec((1, NB), lambda i: (0, i)),
        pl.BlockSpec((G, NB), lambda i: (0, i)),
        pl.BlockSpec((G, NB), lambda i: (0, i)),
        pl.BlockSpec((1, H), lambda i: (0, 0)),
        pl.BlockSpec((HH, OUT), lambda i: (0, 0)),
        pl.BlockSpec((HH, OUT), lambda i: (0, 0)),
        pl.BlockSpec((1, OUT), lambda i: (0, 0)),
        pl.BlockSpec((1, OUT), lambda i: (0, 0)),
    ],
    out_specs=pl.BlockSpec((1, 1), lambda i: (0, 0)),
    out_shape=jax.ShapeDtypeStruct((1, 1), jnp.float32),
    scratch_shapes=[
        pltpu.VMEM((G, H), jnp.float32),
        pltpu.VMEM((G, 1), jnp.float32),
    ],
)


def kernel(x, edge_index, batch, W1g, b1g, W2g, b2g, W1t, b1t, W2t, b2t):
    pad_edges = N + jnp.arange(EPAD - E, dtype=jnp.int32) % (NPAD - N)
    src1 = jnp.concatenate([edge_index[0], pad_edges])
    dst1 = jnp.concatenate([edge_index[1], pad_edges])
    src2 = src1.reshape(NCHUNK, CH)
    dst2 = dst1.reshape(NCHUNK, CH)
    Wcat = jnp.concatenate([W1g, W1t], axis=1)
    b1cat = jnp.concatenate([b1g, b1t]).reshape(1, H)
    ones_ch = jnp.ones((CH,), jnp.float32)
    z1 = jnp.zeros((RPT,), jnp.float32)
    z2 = jnp.zeros((RPT, H), jnp.float32)
    zv = jnp.zeros((VPT,), jnp.float32)
    x_pad = jnp.pad(x, ((0, NPAD - N), (0, 0)))
    batch_pad = jnp.pad(batch, (0, NPAD - N), constant_values=G)

    h = _mm_call(x_pad, Wcat)
    deg_parts = _deg_call(dst2, ones_ch, z1)
    d0 = deg_parts[0].reshape(NPAD, 1)
    d1 = deg_parts[1].reshape(NPAD, 1)
    h2, dinv = _scale_call(h, d0, d1)
    aggp = _agg_call(src1, dst1, h2, z2)
    vp = _vtab_call(src2, dst2, dinv.reshape(NPAD), batch_pad, zv)
    out = _final_call(
        aggp[0], aggp[1], h2, dinv, dinv.reshape(1, NPAD),
        batch_pad.reshape(1, NPAD), vp[0].reshape(G, NPAD),
        vp[1].reshape(G, NPAD), b1cat, W2g, W2t,
        b2g.reshape(1, OUT), b2t.reshape(1, OUT))
    return out.reshape(())


# speedup vs baseline: 1.0768x; 1.0768x over previous
import dataclasses
import functools

import jax
import jax.numpy as jnp
from jax import lax
from jax.experimental import pallas as pl
from jax.experimental.pallas import tpu as pltpu
from jax.experimental.pallas import tpu_sc as plsc

N = 10000
E = 320000
G = 64
CIN = 128
H = 128
HH = 64
OUT = 32
NC, NS = 2, 16
NW = NC * NS
CH = 128
NCHUNK = 2560
EPAD = NCHUNK * CH
C16 = NCHUNK // 16
C32 = NCHUNK // 32
NPAD = 10240
RPT = NPAD // NS
VSIZE = G * NPAD
VPT = VSIZE // NS
NB = 1024

_mesh = plsc.VectorSubcoreMesh(core_axis_name="core", subcore_axis_name="subcore")

_sc_params = pltpu.CompilerParams()
if "needs_layout_passes" in pltpu.CompilerParams.__dataclass_fields__:
    _sc_params = dataclasses.replace(_sc_params, needs_layout_passes=False)


@functools.partial(
    pl.kernel,
    out_type=jax.ShapeDtypeStruct((NC, NPAD), jnp.float32),
    mesh=_mesh,
    scratch_types=[
        pltpu.VMEM((C32, CH), jnp.int32),
        pltpu.VMEM((CH,), jnp.float32),
        pltpu.VMEM_SHARED((NPAD,), jnp.float32),
        pltpu.SemaphoreType.DMA,
    ],
    compiler_params=_sc_params,
)
def _deg_call(dst2_hbm, ones_hbm, z_hbm, deg_out, dstx, onesv, deg_sh, sem):
    c = lax.axis_index("core")
    s = lax.axis_index("subcore")
    w = c * NS + s
    off = w * C32
    pltpu.sync_copy(z_hbm, deg_sh.at[pl.ds(s * RPT, RPT)])
    pltpu.sync_copy(ones_hbm, onesv)
    pltpu.sync_copy(dst2_hbm.at[pl.ds(off, C32)], dstx)
    plsc.subcore_barrier()

    @pl.loop(0, C32)
    def _(j):
        pltpu.async_copy(onesv, deg_sh.at[dstx.at[j]], sem, add=True)

    @pl.loop(0, C32)
    def _(j):
        pltpu.make_async_copy(onesv, deg_sh.at[dstx.at[0]], sem).wait()

    plsc.subcore_barrier()
    pltpu.sync_copy(deg_sh.at[pl.ds(s * RPT, RPT)],
                    deg_out.at[c, pl.ds(s * RPT, RPT)])


def _mm_body(x_ref, w_ref, d0_ref, d1_ref, h2_ref, dinv_ref):
    dinv = lax.rsqrt(d0_ref[...] + d1_ref[...] + 1.0)
    h = jnp.dot(x_ref[...], w_ref[...], preferred_element_type=jnp.float32)
    h2_ref[...] = dinv * h
    dinv_ref[...] = dinv


_mm_call = pl.pallas_call(
    _mm_body,
    grid=(NPAD // NB,),
    in_specs=[
        pl.BlockSpec((NB, CIN), lambda i: (i, 0)),
        pl.BlockSpec((CIN, H), lambda i: (0, 0)),
        pl.BlockSpec((NB, 1), lambda i: (i, 0)),
        pl.BlockSpec((NB, 1), lambda i: (i, 0)),
    ],
    out_specs=[
        pl.BlockSpec((NB, H), lambda i: (i, 0)),
        pl.BlockSpec((NB, 1), lambda i: (i, 0)),
    ],
    out_shape=[
        jax.ShapeDtypeStruct((NPAD, H), jnp.float32),
        jax.ShapeDtypeStruct((NPAD, 1), jnp.float32),
    ],
)


CH1 = 80
K1 = (EPAD // NW) // CH1

@functools.partial(
    pl.kernel,
    out_type=jax.ShapeDtypeStruct((NC, NPAD, H), jnp.float32),
    mesh=_mesh,
    scratch_types=(
        [pltpu.VMEM((CH1,), jnp.int32)] * 16
        + [pltpu.VMEM((CH1, H), jnp.float32)] * 4
        + [pltpu.VMEM_SHARED((NPAD, H), jnp.float32)]
        + [pltpu.SemaphoreType.DMA] * 16
    ),
    compiler_params=_sc_params,
)
def _agg_call(src1_hbm, dst1_hbm, h2_hbm, z2_hbm, agg_out,
              sx0, sx1, sx2, sx3, sx4, sx5, sx6, sx7,
              dx0, dx1, dx2, dx3, dx4, dx5, dx6, dx7,
              r0, r1, r2, r3, agg_sh,
              si0, si1, si2, si3, si4, si5, si6, si7,
              sg0, sg1, sg2, sg3, sc0, sc1, sc2, sc3):
    c = lax.axis_index("core")
    s = lax.axis_index("subcore")
    w = c * NS + s
    base = w * K1 * CH1
    sx = (sx0, sx1, sx2, sx3, sx4, sx5, sx6, sx7)
    dx = (dx0, dx1, dx2, dx3, dx4, dx5, dx6, dx7)
    rows = (r0, r1, r2, r3)
    si = (si0, si1, si2, si3, si4, si5, si6, si7)
    sg = (sg0, sg1, sg2, sg3)
    sc = (sc0, sc1, sc2, sc3)

    def fire_idx(j, q):
        e = pl.multiple_of(base + j * CH1, 8)
        pltpu.async_copy(src1_hbm.at[pl.ds(e, CH1)], sx[q], si[q])
        pltpu.async_copy(dst1_hbm.at[pl.ds(e, CH1)], dx[q], si[q])

    def wait_idx(q):
        pltpu.make_async_copy(src1_hbm.at[pl.ds(0, CH1)], sx[q], si[q]).wait()
        pltpu.make_async_copy(dst1_hbm.at[pl.ds(0, CH1)], dx[q], si[q]).wait()

    def fire_gather(q, b):
        pltpu.async_copy(h2_hbm.at[sx[q]], rows[b], sg[b])

    def wait_gather(b):
        pltpu.make_async_copy(h2_hbm.at[sx[0]], rows[b], sg[b]).wait()

    def fire_scatter(q, b):
        pltpu.async_copy(rows[b], agg_sh.at[dx[q]], sc[b], add=True)

    def wait_scatter(b):
        pltpu.make_async_copy(rows[b], agg_sh.at[dx[0]], sc[b]).wait()

    pltpu.sync_copy(z2_hbm, agg_sh.at[pl.ds(s * RPT, RPT)])
    plsc.subcore_barrier()

    for q in range(6):
        fire_idx(q, q)
    wait_idx(0)
    fire_gather(0, 0)
    wait_idx(1)
    fire_gather(1, 1)

    @pl.loop(0, K1 // 8)
    def _(i):
        for u in range(8):
            b = u % 4
            b2 = (u + 2) % 4
            q = u
            q2 = (u + 2) % 8
            q6 = (u + 6) % 8
            wait_gather(b)
            fire_scatter(q, b)
            gate2 = pl.when(i >= 1) if u < 2 else (lambda f: f())
            gate_hi = (lambda f: f()) if u < 6 else pl.when(i < K1 // 8 - 1)
            gate_hi2 = (lambda f: f()) if u < 2 else pl.when(i < K1 // 8 - 1)

            @gate2
            def _():
                wait_scatter(b2)

            @gate_hi
            def _():
                wait_idx(q2)
                fire_gather(q2, b2)

            @gate_hi2
            def _():
                fire_idx(8 * i + u + 6, q6)

    wait_scatter(2)
    wait_scatter(3)
    plsc.subcore_barrier()
    pltpu.sync_copy(agg_sh.at[pl.ds(s * RPT, RPT)],
                    agg_out.at[c, pl.ds(s * RPT, RPT)])


@functools.partial(
    pl.kernel,
    out_type=jax.ShapeDtypeStruct((NC, VSIZE), jnp.float32),
    mesh=_mesh,
    scratch_types=[
        pltpu.VMEM((C32, CH), jnp.int32),
        pltpu.VMEM((C32, CH), jnp.int32),
        pltpu.VMEM((C32, CH), jnp.float32),
        pltpu.VMEM((C32, CH), jnp.int32),
        pltpu.VMEM((NPAD,), jnp.float32),
        pltpu.VMEM((NPAD,), jnp.int32),
        pltpu.VMEM_SHARED((VSIZE,), jnp.float32),
        pltpu.SemaphoreType.DMA,
    ],
    compiler_params=_sc_params,
)
def _vtab_call(src2_hbm, dst2_hbm, dinv_hbm, batch_hbm, z1_hbm, v_out,
               srcx, dstx, vvals, vidx, dinvv, batchv, v_sh, sem):
    c = lax.axis_index("core")
    s = lax.axis_index("subcore")
    w = c * NS + s
    off = w * C32
    pltpu.sync_copy(z1_hbm, v_sh.at[pl.ds(s * VPT, VPT)])
    pltpu.sync_copy(dinv_hbm, dinvv)
    pltpu.sync_copy(batch_hbm, batchv)
    pltpu.sync_copy(src2_hbm.at[pl.ds(off, C32)], srcx)
    pltpu.sync_copy(dst2_hbm.at[pl.ds(off, C32)], dstx)

    zf16 = jnp.zeros((16,), jnp.float32)
    zi16 = jnp.zeros((16,), jnp.int32)
    for k in range((NPAD - N) // 16):
        dinvv[pl.ds(N + 16 * k, 16)] = zf16
        batchv[pl.ds(N + 16 * k, 16)] = zi16
    plsc.subcore_barrier()

    @pl.loop(0, C32)
    def _(j):
        @pl.loop(0, CH // 16)
        def _(k):
            s16 = srcx[j, pl.ds(16 * k, 16)]
            d16 = dstx[j, pl.ds(16 * k, 16)]
            vvals[j, pl.ds(16 * k, 16)] = plsc.load_gather(dinvv, [d16])
            vidx[j, pl.ds(16 * k, 16)] = (
                plsc.load_gather(batchv, [d16]) * NPAD + s16)

    @pl.loop(0, C32)
    def _(j):
        pltpu.async_copy(vvals.at[j], v_sh.at[vidx.at[j]], sem, add=True)

    @pl.loop(0, C32)
    def _(j):
        pltpu.make_async_copy(vvals.at[0], v_sh.at[vidx.at[0]], sem).wait()

    plsc.subcore_barrier()
    pltpu.sync_copy(v_sh.at[pl.ds(s * VPT, VPT)],
                    v_out.at[c, pl.ds(s * VPT, VPT)])


def _final_body(a0_ref, a1_ref, h2_ref, dcol_ref, drow_ref, brow_ref, v0_ref,
                v1_ref, b1_ref, w2g_ref, w2t_ref, b2g_ref, b2t_ref, out_ref,
                u_acc, cnt_acc):
    i = pl.program_id(0)

    @pl.when(i == 0)
    def _():
        u_acc[...] = jnp.zeros_like(u_acc)
        cnt_acc[...] = jnp.zeros_like(cnt_acc)

    dinv = dcol_ref[...]
    a = a0_ref[...] + a1_ref[...] + h2_ref[...]
    rd = dinv * jnp.maximum(dinv * a + b1_ref[...], 0.0)
    gids = lax.broadcasted_iota(jnp.int32, (G, NB), 0)
    onehot = brow_ref[...] == gids
    v_eff = v0_ref[...] + v1_ref[...] + jnp.where(onehot, drow_ref[...], 0.0)
    u_acc[...] += jnp.dot(v_eff, rd, preferred_element_type=jnp.float32)
    cnt_acc[...] += jnp.sum(onehot.astype(jnp.float32), axis=1, keepdims=True)

    @pl.when(i == pl.num_programs(0) - 1)
    def _():
        cnt = cnt_acc[...]
        cinv = 1.0 / jnp.maximum(cnt, 1.0)
        nz = jnp.where(cnt > 0, 1.0, 0.0)
        us = u_acc[...] * cinv
        pg = jnp.dot(us[:, :HH], w2g_ref[...],
                     preferred_element_type=jnp.float32) + b2g_ref[...] * nz
        pt = jnp.dot(us[:, HH:], w2t_ref[...],
                     preferred_element_type=jnp.float32) + b2t_ref[...] * nz
        diff = pt - pg + 1e-6
        dist = jnp.sqrt(jnp.sum(diff * diff, axis=1, keepdims=True))
        out_ref[...] = jnp.sum(dist).reshape(1, 1) / G


_final_call = pl.pallas_call(
    _final_body,
    grid=(NPAD // NB,),
    in_specs=[
        pl.BlockSpec((NB, H), lambda i: (i, 0)),
        pl.BlockSpec((NB, H), lambda i: (i, 0)),
        pl.BlockSpec((NB, H), lambda i: (i, 0)),
        pl.BlockSpec((NB, 1), lambda i: (i, 0)),
        pl.BlockSpec((1, NB), lambda i: (0, i)),
        pl.BlockSpec((1, NB), lambda i: (0, i)),
        pl.BlockSpec((G, NB), lambda i: (0, i)),
        pl.BlockSpec((G, NB), lambda i: (0, i)),
        pl.BlockSpec((1, H), lambda i: (0, 0)),
        pl.BlockSpec((HH, OUT), lambda i: (0, 0)),
        pl.BlockSpec((HH, OUT), lambda i: (0, 0)),
        pl.BlockSpec((1, OUT), lambda i: (0, 0)),
        pl.BlockSpec((1, OUT), lambda i: (0, 0)),
    ],
    out_specs=pl.BlockSpec((1, 1), lambda i: (0, 0)),
    out_shape=jax.ShapeDtypeStruct((1, 1), jnp.float32),
    scratch_shapes=[
        pltpu.VMEM((G, H), jnp.float32),
        pltpu.VMEM((G, 1), jnp.float32),
    ],
)


def kernel(x, edge_index, batch, W1g, b1g, W2g, b2g, W1t, b1t, W2t, b2t):
    pad_edges = N + jnp.arange(EPAD - E, dtype=jnp.int32) % (NPAD - N)
    src1 = jnp.concatenate([edge_index[0], pad_edges])
    dst1 = jnp.concatenate([edge_index[1], pad_edges])
    src2 = src1.reshape(NCHUNK, CH)
    dst2 = dst1.reshape(NCHUNK, CH)
    Wcat = jnp.concatenate([W1g, W1t], axis=1)
    b1cat = jnp.concatenate([b1g, b1t]).reshape(1, H)
    ones_ch = jnp.ones((CH,), jnp.float32)
    z1 = jnp.zeros((RPT,), jnp.float32)
    z2 = jnp.zeros((RPT, H), jnp.float32)
    zv = jnp.zeros((VPT,), jnp.float32)
    x_pad = jnp.pad(x, ((0, NPAD - N), (0, 0)))
    batch_pad = jnp.pad(batch, (0, NPAD - N), constant_values=G)

    deg_parts = _deg_call(dst2, ones_ch, z1)
    d0 = deg_parts[0].reshape(NPAD, 1)
    d1 = deg_parts[1].reshape(NPAD, 1)
    h2, dinv = _mm_call(x_pad, Wcat, d0, d1)
    aggp = _agg_call(src1, dst1, h2, z2)
    vp = _vtab_call(src2, dst2, dinv.reshape(NPAD), batch_pad, zv)
    out = _final_call(
        aggp[0], aggp[1], h2, dinv, dinv.reshape(1, NPAD),
        batch_pad.reshape(1, NPAD), vp[0].reshape(G, NPAD),
        vp[1].reshape(G, NPAD), b1cat, W2g, W2t,
        b2g.reshape(1, OUT), b2t.reshape(1, OUT))
    return out.reshape(())


# speedup vs baseline: 1.0981x; 1.0198x over previous
import dataclasses
import functools

import jax
import jax.numpy as jnp
from jax import lax
from jax.experimental import pallas as pl
from jax.experimental.pallas import tpu as pltpu
from jax.experimental.pallas import tpu_sc as plsc

N = 10000
E = 320000
G = 64
CIN = 128
H = 128
HH = 64
OUT = 32
NC, NS = 2, 16
NW = NC * NS
CH = 128
NCHUNK = 2560
EPAD = NCHUNK * CH
C16 = NCHUNK // 16
C32 = NCHUNK // 32
NPAD = 10240
RPT = NPAD // NS
VSIZE = G * NPAD
VPT = VSIZE // NS
NB = 1024

_mesh = plsc.VectorSubcoreMesh(core_axis_name="core", subcore_axis_name="subcore")

_sc_params = pltpu.CompilerParams()
if "needs_layout_passes" in pltpu.CompilerParams.__dataclass_fields__:
    _sc_params = dataclasses.replace(_sc_params, needs_layout_passes=False)


@functools.partial(
    pl.kernel,
    out_type=jax.ShapeDtypeStruct((NC, NPAD), jnp.float32),
    mesh=_mesh,
    scratch_types=[
        pltpu.VMEM((C32, CH), jnp.int32),
        pltpu.VMEM((CH,), jnp.float32),
        pltpu.VMEM_SHARED((NPAD,), jnp.float32),
        pltpu.SemaphoreType.DMA,
    ],
    compiler_params=_sc_params,
)
def _deg_call(dst2_hbm, ones_hbm, z_hbm, deg_out, dstx, onesv, deg_sh, sem):
    c = lax.axis_index("core")
    s = lax.axis_index("subcore")
    w = c * NS + s
    off = w * C32
    pltpu.async_copy(z_hbm, deg_sh.at[pl.ds(s * RPT, RPT)], sem)
    pltpu.async_copy(ones_hbm, onesv, sem)
    pltpu.async_copy(dst2_hbm.at[pl.ds(off, C32)], dstx, sem)
    pltpu.make_async_copy(z_hbm, deg_sh.at[pl.ds(s * RPT, RPT)], sem).wait()
    pltpu.make_async_copy(ones_hbm, onesv, sem).wait()
    pltpu.make_async_copy(dst2_hbm.at[pl.ds(off, C32)], dstx, sem).wait()
    plsc.subcore_barrier()

    @pl.loop(0, C32)
    def _(j):
        pltpu.async_copy(onesv, deg_sh.at[dstx.at[j]], sem, add=True)

    @pl.loop(0, C32)
    def _(j):
        pltpu.make_async_copy(onesv, deg_sh.at[dstx.at[0]], sem).wait()

    plsc.subcore_barrier()
    pltpu.sync_copy(deg_sh.at[pl.ds(s * RPT, RPT)],
                    deg_out.at[c, pl.ds(s * RPT, RPT)])


def _mm_body(x_ref, w_ref, d0_ref, d1_ref, h2_ref, dinv_ref):
    dinv = lax.rsqrt(d0_ref[...] + d1_ref[...] + 1.0)
    h = jnp.dot(x_ref[...], w_ref[...], preferred_element_type=jnp.float32)
    h2_ref[...] = dinv * h
    dinv_ref[...] = dinv


_mm_call = pl.pallas_call(
    _mm_body,
    grid=(NPAD // NB,),
    in_specs=[
        pl.BlockSpec((NB, CIN), lambda i: (i, 0)),
        pl.BlockSpec((CIN, H), lambda i: (0, 0)),
        pl.BlockSpec((NB, 1), lambda i: (i, 0)),
        pl.BlockSpec((NB, 1), lambda i: (i, 0)),
    ],
    out_specs=[
        pl.BlockSpec((NB, H), lambda i: (i, 0)),
        pl.BlockSpec((NB, 1), lambda i: (i, 0)),
    ],
    out_shape=[
        jax.ShapeDtypeStruct((NPAD, H), jnp.float32),
        jax.ShapeDtypeStruct((NPAD, 1), jnp.float32),
    ],
)


CH1 = 80
K1 = (EPAD // NW) // CH1

@functools.partial(
    pl.kernel,
    out_type=jax.ShapeDtypeStruct((NC, NPAD, H), jnp.float32),
    mesh=_mesh,
    scratch_types=(
        [pltpu.VMEM((CH1,), jnp.int32)] * 16
        + [pltpu.VMEM((CH1, H), jnp.float32)] * 4
        + [pltpu.VMEM_SHARED((NPAD, H), jnp.float32)]
        + [pltpu.SemaphoreType.DMA] * 16
    ),
    compiler_params=_sc_params,
)
def _agg_call(src1_hbm, dst1_hbm, h2_hbm, z2_hbm, agg_out,
              sx0, sx1, sx2, sx3, sx4, sx5, sx6, sx7,
              dx0, dx1, dx2, dx3, dx4, dx5, dx6, dx7,
              r0, r1, r2, r3, agg_sh,
              si0, si1, si2, si3, si4, si5, si6, si7,
              sg0, sg1, sg2, sg3, sc0, sc1, sc2, sc3):
    c = lax.axis_index("core")
    s = lax.axis_index("subcore")
    w = c * NS + s
    base = w * K1 * CH1
    sx = (sx0, sx1, sx2, sx3, sx4, sx5, sx6, sx7)
    dx = (dx0, dx1, dx2, dx3, dx4, dx5, dx6, dx7)
    rows = (r0, r1, r2, r3)
    si = (si0, si1, si2, si3, si4, si5, si6, si7)
    sg = (sg0, sg1, sg2, sg3)
    sc = (sc0, sc1, sc2, sc3)

    def fire_idx(j, q):
        e = pl.multiple_of(base + j * CH1, 8)
        pltpu.async_copy(src1_hbm.at[pl.ds(e, CH1)], sx[q], si[q])
        pltpu.async_copy(dst1_hbm.at[pl.ds(e, CH1)], dx[q], si[q])

    def wait_idx(q):
        pltpu.make_async_copy(src1_hbm.at[pl.ds(0, CH1)], sx[q], si[q]).wait()
        pltpu.make_async_copy(dst1_hbm.at[pl.ds(0, CH1)], dx[q], si[q]).wait()

    def fire_gather(q, b):
        pltpu.async_copy(h2_hbm.at[sx[q]], rows[b], sg[b])

    def wait_gather(b):
        pltpu.make_async_copy(h2_hbm.at[sx[0]], rows[b], sg[b]).wait()

    def fire_scatter(q, b):
        pltpu.async_copy(rows[b], agg_sh.at[dx[q]], sc[b], add=True)

    def wait_scatter(b):
        pltpu.make_async_copy(rows[b], agg_sh.at[dx[0]], sc[b]).wait()

    pltpu.sync_copy(z2_hbm, agg_sh.at[pl.ds(s * RPT, RPT)])
    plsc.subcore_barrier()

    for q in range(6):
        fire_idx(q, q)
    wait_idx(0)
    fire_gather(0, 0)
    wait_idx(1)
    fire_gather(1, 1)

    @pl.loop(0, K1 // 8)
    def _(i):
        for u in range(8):
            b = u % 4
            b2 = (u + 2) % 4
            q = u
            q2 = (u + 2) % 8
            q6 = (u + 6) % 8
            wait_gather(b)
            fire_scatter(q, b)
            gate2 = pl.when(i >= 1) if u < 2 else (lambda f: f())
            gate_hi = (lambda f: f()) if u < 6 else pl.when(i < K1 // 8 - 1)
            gate_hi2 = (lambda f: f()) if u < 2 else pl.when(i < K1 // 8 - 1)

            @gate2
            def _():
                wait_scatter(b2)

            @gate_hi
            def _():
                wait_idx(q2)
                fire_gather(q2, b2)

            @gate_hi2
            def _():
                fire_idx(8 * i + u + 6, q6)

    wait_scatter(2)
    wait_scatter(3)
    plsc.subcore_barrier()
    pltpu.sync_copy(agg_sh.at[pl.ds(s * RPT, RPT)],
                    agg_out.at[c, pl.ds(s * RPT, RPT)])


@functools.partial(
    pl.kernel,
    out_type=jax.ShapeDtypeStruct((NC, VSIZE), jnp.float32),
    mesh=_mesh,
    scratch_types=[
        pltpu.VMEM((C32, CH), jnp.int32),
        pltpu.VMEM((C32, CH), jnp.int32),
        pltpu.VMEM((C32, CH), jnp.float32),
        pltpu.VMEM((C32, CH), jnp.int32),
        pltpu.VMEM((NPAD,), jnp.float32),
        pltpu.VMEM((NPAD,), jnp.int32),
        pltpu.VMEM_SHARED((VSIZE,), jnp.float32),
        pltpu.SemaphoreType.DMA,
    ],
    compiler_params=_sc_params,
)
def _vtab_call(src2_hbm, dst2_hbm, dinv_hbm, batch_hbm, z1_hbm, v_out,
               srcx, dstx, vvals, vidx, dinvv, batchv, v_sh, sem):
    c = lax.axis_index("core")
    s = lax.axis_index("subcore")
    w = c * NS + s
    off = w * C32
    pltpu.async_copy(z1_hbm, v_sh.at[pl.ds(s * VPT, VPT)], sem)
    pltpu.async_copy(dinv_hbm, dinvv, sem)
    pltpu.async_copy(batch_hbm, batchv, sem)
    pltpu.async_copy(src2_hbm.at[pl.ds(off, C32)], srcx, sem)
    pltpu.async_copy(dst2_hbm.at[pl.ds(off, C32)], dstx, sem)
    pltpu.make_async_copy(z1_hbm, v_sh.at[pl.ds(s * VPT, VPT)], sem).wait()
    pltpu.make_async_copy(dinv_hbm, dinvv, sem).wait()
    pltpu.make_async_copy(batch_hbm, batchv, sem).wait()
    pltpu.make_async_copy(src2_hbm.at[pl.ds(off, C32)], srcx, sem).wait()
    pltpu.make_async_copy(dst2_hbm.at[pl.ds(off, C32)], dstx, sem).wait()

    zf16 = jnp.zeros((16,), jnp.float32)
    zi16 = jnp.zeros((16,), jnp.int32)
    for k in range((NPAD - N) // 16):
        dinvv[pl.ds(N + 16 * k, 16)] = zf16
        batchv[pl.ds(N + 16 * k, 16)] = zi16
    plsc.subcore_barrier()

    @pl.loop(0, C32)
    def _(j):
        @pl.loop(0, CH // 16)
        def _(k):
            s16 = srcx[j, pl.ds(16 * k, 16)]
            d16 = dstx[j, pl.ds(16 * k, 16)]
            vvals[j, pl.ds(16 * k, 16)] = plsc.load_gather(dinvv, [d16])
            vidx[j, pl.ds(16 * k, 16)] = (
                plsc.load_gather(batchv, [d16]) * NPAD + s16)

    @pl.loop(0, C32)
    def _(j):
        pltpu.async_copy(vvals.at[j], v_sh.at[vidx.at[j]], sem, add=True)

    @pl.loop(0, C32)
    def _(j):
        pltpu.make_async_copy(vvals.at[0], v_sh.at[vidx.at[0]], sem).wait()

    plsc.subcore_barrier()
    pltpu.sync_copy(v_sh.at[pl.ds(s * VPT, VPT)],
                    v_out.at[c, pl.ds(s * VPT, VPT)])


def _final_body(a0_ref, a1_ref, h2_ref, dcol_ref, drow_ref, brow_ref, v0_ref,
                v1_ref, b1_ref, w2g_ref, w2t_ref, b2g_ref, b2t_ref, out_ref,
                u_acc, cnt_acc):
    i = pl.program_id(0)

    @pl.when(i == 0)
    def _():
        u_acc[...] = jnp.zeros_like(u_acc)
        cnt_acc[...] = jnp.zeros_like(cnt_acc)

    dinv = dcol_ref[...]
    a = a0_ref[...] + a1_ref[...] + h2_ref[...]
    rd = dinv * jnp.maximum(dinv * a + b1_ref[...], 0.0)
    gids = lax.broadcasted_iota(jnp.int32, (G, NB), 0)
    onehot = brow_ref[...] == gids
    v_eff = v0_ref[...] + v1_ref[...] + jnp.where(onehot, drow_ref[...], 0.0)
    u_acc[...] += jnp.dot(v_eff, rd, preferred_element_type=jnp.float32)
    cnt_acc[...] += jnp.sum(onehot.astype(jnp.float32), axis=1, keepdims=True)

    @pl.when(i == pl.num_programs(0) - 1)
    def _():
        cnt = cnt_acc[...]
        cinv = 1.0 / jnp.maximum(cnt, 1.0)
        nz = jnp.where(cnt > 0, 1.0, 0.0)
        us = u_acc[...] * cinv
        pg = jnp.dot(us[:, :HH], w2g_ref[...],
                     preferred_element_type=jnp.float32) + b2g_ref[...] * nz
        pt = jnp.dot(us[:, HH:], w2t_ref[...],
                     preferred_element_type=jnp.float32) + b2t_ref[...] * nz
        diff = pt - pg + 1e-6
        dist = jnp.sqrt(jnp.sum(diff * diff, axis=1, keepdims=True))
        out_ref[...] = jnp.sum(dist).reshape(1, 1) / G


_final_call = pl.pallas_call(
    _final_body,
    grid=(NPAD // NB,),
    in_specs=[
        pl.BlockSpec((NB, H), lambda i: (i, 0)),
        pl.BlockSpec((NB, H), lambda i: (i, 0)),
        pl.BlockSpec((NB, H), lambda i: (i, 0)),
        pl.BlockSpec((NB, 1), lambda i: (i, 0)),
        pl.BlockSpec((1, NB), lambda i: (0, i)),
        pl.BlockSpec((1, NB), lambda i: (0, i)),
        pl.BlockSpec((G, NB), lambda i: (0, i)),
        pl.BlockSpec((G, NB), lambda i: (0, i)),
        pl.BlockSpec((1, H), lambda i: (0, 0)),
        pl.BlockSpec((HH, OUT), lambda i: (0, 0)),
        pl.BlockSpec((HH, OUT), lambda i: (0, 0)),
        pl.BlockSpec((1, OUT), lambda i: (0, 0)),
        pl.BlockSpec((1, OUT), lambda i: (0, 0)),
    ],
    out_specs=pl.BlockSpec((1, 1), lambda i: (0, 0)),
    out_shape=jax.ShapeDtypeStruct((1, 1), jnp.float32),
    scratch_shapes=[
        pltpu.VMEM((G, H), jnp.float32),
        pltpu.VMEM((G, 1), jnp.float32),
    ],
)


def kernel(x, edge_index, batch, W1g, b1g, W2g, b2g, W1t, b1t, W2t, b2t):
    pad_edges = N + jnp.arange(EPAD - E, dtype=jnp.int32) % (NPAD - N)
    src1 = jnp.concatenate([edge_index[0], pad_edges])
    dst1 = jnp.concatenate([edge_index[1], pad_edges])
    src2 = src1.reshape(NCHUNK, CH)
    dst2 = dst1.reshape(NCHUNK, CH)
    Wcat = jnp.concatenate([W1g, W1t], axis=1)
    b1cat = jnp.concatenate([b1g, b1t]).reshape(1, H)
    ones_ch = jnp.ones((CH,), jnp.float32)
    z1 = jnp.zeros((RPT,), jnp.float32)
    z2 = jnp.zeros((RPT, H), jnp.float32)
    zv = jnp.zeros((VPT,), jnp.float32)
    x_pad = jnp.pad(x, ((0, NPAD - N), (0, 0)))
    batch_pad = jnp.pad(batch, (0, NPAD - N), constant_values=G)

    deg_parts = _deg_call(dst2, ones_ch, z1)
    d0 = deg_parts[0].reshape(NPAD, 1)
    d1 = deg_parts[1].reshape(NPAD, 1)
    h2, dinv = _mm_call(x_pad, Wcat, d0, d1)
    aggp = _agg_call(src1, dst1, h2, z2)
    vp = _vtab_call(src2, dst2, dinv.reshape(NPAD), batch_pad, zv)
    out = _final_call(
        aggp[0], aggp[1], h2, dinv, dinv.reshape(1, NPAD),
        batch_pad.reshape(1, NPAD), vp[0].reshape(G, NPAD),
        vp[1].reshape(G, NPAD), b1cat, W2g, W2t,
        b2g.reshape(1, OUT), b2t.reshape(1, OUT))
    return out.reshape(())


# speedup vs baseline: 1.0997x; 1.0014x over previous
import dataclasses
import functools

import jax
import jax.numpy as jnp
from jax import lax
from jax.experimental import pallas as pl
from jax.experimental.pallas import tpu as pltpu
from jax.experimental.pallas import tpu_sc as plsc

N = 10000
E = 320000
G = 64
CIN = 128
H = 128
HH = 64
OUT = 32
NC, NS = 2, 16
NW = NC * NS
CH = 128
NCHUNK = 2560
EPAD = NCHUNK * CH
C16 = NCHUNK // 16
C32 = NCHUNK // 32
NPAD = 10240
RPT = NPAD // NS
VSIZE = G * NPAD
VPT = VSIZE // NS
NB = 1024

_mesh = plsc.VectorSubcoreMesh(core_axis_name="core", subcore_axis_name="subcore")

_sc_params = pltpu.CompilerParams()
if "needs_layout_passes" in pltpu.CompilerParams.__dataclass_fields__:
    _sc_params = dataclasses.replace(_sc_params, needs_layout_passes=False)


@functools.partial(
    pl.kernel,
    out_type=jax.ShapeDtypeStruct((NC, NPAD), jnp.float32),
    mesh=_mesh,
    scratch_types=[
        pltpu.VMEM((C32, CH), jnp.int32),
        pltpu.VMEM((CH,), jnp.float32),
        pltpu.VMEM_SHARED((NPAD,), jnp.float32),
        pltpu.SemaphoreType.DMA,
    ],
    compiler_params=_sc_params,
)
def _deg_call(dst2_hbm, ones_hbm, z_hbm, deg_out, dstx, onesv, deg_sh, sem):
    c = lax.axis_index("core")
    s = lax.axis_index("subcore")
    w = c * NS + s
    off = w * C32
    pltpu.async_copy(z_hbm, deg_sh.at[pl.ds(s * RPT, RPT)], sem)
    pltpu.async_copy(ones_hbm, onesv, sem)
    pltpu.async_copy(dst2_hbm.at[pl.ds(off, C32)], dstx, sem)
    pltpu.make_async_copy(z_hbm, deg_sh.at[pl.ds(s * RPT, RPT)], sem).wait()
    pltpu.make_async_copy(ones_hbm, onesv, sem).wait()
    pltpu.make_async_copy(dst2_hbm.at[pl.ds(off, C32)], dstx, sem).wait()
    plsc.subcore_barrier()

    @pl.loop(0, C32)
    def _(j):
        pltpu.async_copy(onesv, deg_sh.at[dstx.at[j]], sem, add=True)

    @pl.loop(0, C32)
    def _(j):
        pltpu.make_async_copy(onesv, deg_sh.at[dstx.at[0]], sem).wait()

    plsc.subcore_barrier()
    pltpu.sync_copy(deg_sh.at[pl.ds(s * RPT, RPT)],
                    deg_out.at[c, pl.ds(s * RPT, RPT)])


def _mm_body(x_ref, w_ref, d0_ref, d1_ref, h2_ref, dinv_ref):
    dinv = lax.rsqrt(d0_ref[...] + d1_ref[...] + 1.0)
    h = jnp.dot(x_ref[...], w_ref[...], preferred_element_type=jnp.float32)
    h2_ref[...] = dinv * h
    dinv_ref[...] = dinv


_mm_call = pl.pallas_call(
    _mm_body,
    grid=(NPAD // NB,),
    in_specs=[
        pl.BlockSpec((NB, CIN), lambda i: (i, 0)),
        pl.BlockSpec((CIN, H), lambda i: (0, 0)),
        pl.BlockSpec((NB, 1), lambda i: (i, 0)),
        pl.BlockSpec((NB, 1), lambda i: (i, 0)),
    ],
    out_specs=[
        pl.BlockSpec((NB, H), lambda i: (i, 0)),
        pl.BlockSpec((NB, 1), lambda i: (i, 0)),
    ],
    out_shape=[
        jax.ShapeDtypeStruct((NPAD, H), jnp.float32),
        jax.ShapeDtypeStruct((NPAD, 1), jnp.float32),
    ],
)


CH1 = 80
K1 = (EPAD // NW) // CH1

@functools.partial(
    pl.kernel,
    out_type=jax.ShapeDtypeStruct((NC, NPAD, H), jnp.float32),
    mesh=_mesh,
    scratch_types=(
        [pltpu.VMEM((CH1,), jnp.int32)] * 16
        + [pltpu.VMEM((CH1, H), jnp.float32)] * 4
        + [pltpu.VMEM_SHARED((NPAD, H), jnp.float32)]
        + [pltpu.SemaphoreType.DMA] * 16
    ),
    compiler_params=_sc_params,
)
def _agg_call(src1_hbm, dst1_hbm, h2_hbm, z2_hbm, agg_out,
              sx0, sx1, sx2, sx3, sx4, sx5, sx6, sx7,
              dx0, dx1, dx2, dx3, dx4, dx5, dx6, dx7,
              r0, r1, r2, r3, agg_sh,
              si0, si1, si2, si3, si4, si5, si6, si7,
              sg0, sg1, sg2, sg3, sc0, sc1, sc2, sc3):
    c = lax.axis_index("core")
    s = lax.axis_index("subcore")
    w = c * NS + s
    base = w * K1 * CH1
    sx = (sx0, sx1, sx2, sx3, sx4, sx5, sx6, sx7)
    dx = (dx0, dx1, dx2, dx3, dx4, dx5, dx6, dx7)
    rows = (r0, r1, r2, r3)
    si = (si0, si1, si2, si3, si4, si5, si6, si7)
    sg = (sg0, sg1, sg2, sg3)
    sc = (sc0, sc1, sc2, sc3)

    def fire_idx(j, q):
        e = pl.multiple_of(base + j * CH1, 8)
        pltpu.async_copy(src1_hbm.at[pl.ds(e, CH1)], sx[q], si[q])
        pltpu.async_copy(dst1_hbm.at[pl.ds(e, CH1)], dx[q], si[q])

    def wait_idx(q):
        pltpu.make_async_copy(src1_hbm.at[pl.ds(0, CH1)], sx[q], si[q]).wait()
        pltpu.make_async_copy(dst1_hbm.at[pl.ds(0, CH1)], dx[q], si[q]).wait()

    def fire_gather(q, b):
        pltpu.async_copy(h2_hbm.at[sx[q]], rows[b], sg[b])

    def wait_gather(b):
        pltpu.make_async_copy(h2_hbm.at[sx[0]], rows[b], sg[b]).wait()

    def fire_scatter(q, b):
        pltpu.async_copy(rows[b], agg_sh.at[dx[q]], sc[b], add=True)

    def wait_scatter(b):
        pltpu.make_async_copy(rows[b], agg_sh.at[dx[0]], sc[b]).wait()

    pltpu.async_copy(z2_hbm, agg_sh.at[pl.ds(s * RPT, RPT)], sc0)
    for q in range(6):
        fire_idx(q, q)
    pltpu.make_async_copy(z2_hbm, agg_sh.at[pl.ds(s * RPT, RPT)], sc0).wait()
    plsc.subcore_barrier()

    wait_idx(0)
    fire_gather(0, 0)
    wait_idx(1)
    fire_gather(1, 1)

    @pl.loop(0, K1 // 8)
    def _(i):
        for u in range(8):
            b = u % 4
            b2 = (u + 2) % 4
            q = u
            q2 = (u + 2) % 8
            q6 = (u + 6) % 8
            wait_gather(b)
            fire_scatter(q, b)
            gate2 = pl.when(i >= 1) if u < 2 else (lambda f: f())
            gate_hi = (lambda f: f()) if u < 6 else pl.when(i < K1 // 8 - 1)
            gate_hi2 = (lambda f: f()) if u < 2 else pl.when(i < K1 // 8 - 1)

            @gate2
            def _():
                wait_scatter(b2)

            @gate_hi
            def _():
                wait_idx(q2)
                fire_gather(q2, b2)

            @gate_hi2
            def _():
                fire_idx(8 * i + u + 6, q6)

    wait_scatter(2)
    wait_scatter(3)
    plsc.subcore_barrier()
    pltpu.sync_copy(agg_sh.at[pl.ds(s * RPT, RPT)],
                    agg_out.at[c, pl.ds(s * RPT, RPT)])


@functools.partial(
    pl.kernel,
    out_type=jax.ShapeDtypeStruct((NC, VSIZE), jnp.float32),
    mesh=_mesh,
    scratch_types=[
        pltpu.VMEM((C32, CH), jnp.int32),
        pltpu.VMEM((C32, CH), jnp.int32),
        pltpu.VMEM((C32, CH), jnp.float32),
        pltpu.VMEM((C32, CH), jnp.int32),
        pltpu.VMEM((NPAD,), jnp.float32),
        pltpu.VMEM((NPAD,), jnp.int32),
        pltpu.VMEM_SHARED((VSIZE,), jnp.float32),
        pltpu.SemaphoreType.DMA,
    ],
    compiler_params=_sc_params,
)
def _vtab_call(src2_hbm, dst2_hbm, dinv_hbm, batch_hbm, z1_hbm, v_out,
               srcx, dstx, vvals, vidx, dinvv, batchv, v_sh, sem):
    c = lax.axis_index("core")
    s = lax.axis_index("subcore")
    w = c * NS + s
    off = w * C32
    pltpu.async_copy(z1_hbm, v_sh.at[pl.ds(s * VPT, VPT)], sem)
    pltpu.async_copy(dinv_hbm, dinvv, sem)
    pltpu.async_copy(batch_hbm, batchv, sem)
    pltpu.async_copy(src2_hbm.at[pl.ds(off, C32)], srcx, sem)
    pltpu.async_copy(dst2_hbm.at[pl.ds(off, C32)], dstx, sem)
    pltpu.make_async_copy(z1_hbm, v_sh.at[pl.ds(s * VPT, VPT)], sem).wait()
    pltpu.make_async_copy(dinv_hbm, dinvv, sem).wait()
    pltpu.make_async_copy(batch_hbm, batchv, sem).wait()
    pltpu.make_async_copy(src2_hbm.at[pl.ds(off, C32)], srcx, sem).wait()
    pltpu.make_async_copy(dst2_hbm.at[pl.ds(off, C32)], dstx, sem).wait()

    zf16 = jnp.zeros((16,), jnp.float32)
    zi16 = jnp.zeros((16,), jnp.int32)
    for k in range((NPAD - N) // 16):
        dinvv[pl.ds(N + 16 * k, 16)] = zf16
        batchv[pl.ds(N + 16 * k, 16)] = zi16
    plsc.subcore_barrier()

    @pl.loop(0, C32)
    def _(j):
        @pl.loop(0, CH // 16)
        def _(k):
            s16 = srcx[j, pl.ds(16 * k, 16)]
            d16 = dstx[j, pl.ds(16 * k, 16)]
            vvals[j, pl.ds(16 * k, 16)] = plsc.load_gather(dinvv, [d16])
            vidx[j, pl.ds(16 * k, 16)] = (
                plsc.load_gather(batchv, [d16]) * NPAD + s16)

    @pl.loop(0, C32)
    def _(j):
        pltpu.async_copy(vvals.at[j], v_sh.at[vidx.at[j]], sem, add=True)

    @pl.loop(0, C32)
    def _(j):
        pltpu.make_async_copy(vvals.at[0], v_sh.at[vidx.at[0]], sem).wait()

    plsc.subcore_barrier()
    pltpu.sync_copy(v_sh.at[pl.ds(s * VPT, VPT)],
                    v_out.at[c, pl.ds(s * VPT, VPT)])


def _final_body(a0_ref, a1_ref, h2_ref, dcol_ref, drow_ref, brow_ref, v0_ref,
                v1_ref, b1_ref, w2g_ref, w2t_ref, b2g_ref, b2t_ref, out_ref,
                u_acc, cnt_acc):
    i = pl.program_id(0)

    @pl.when(i == 0)
    def _():
        u_acc[...] = jnp.zeros_like(u_acc)
        cnt_acc[...] = jnp.zeros_like(cnt_acc)

    dinv = dcol_ref[...]
    a = a0_ref[...] + a1_ref[...] + h2_ref[...]
    rd = dinv * jnp.maximum(dinv * a + b1_ref[...], 0.0)
    gids = lax.broadcasted_iota(jnp.int32, (G, NB), 0)
    onehot = brow_ref[...] == gids
    v_eff = v0_ref[...] + v1_ref[...] + jnp.where(onehot, drow_ref[...], 0.0)
    u_acc[...] += jnp.dot(v_eff, rd, preferred_element_type=jnp.float32)
    cnt_acc[...] += jnp.sum(onehot.astype(jnp.float32), axis=1, keepdims=True)

    @pl.when(i == pl.num_programs(0) - 1)
    def _():
        cnt = cnt_acc[...]
        cinv = 1.0 / jnp.maximum(cnt, 1.0)
        nz = jnp.where(cnt > 0, 1.0, 0.0)
        us = u_acc[...] * cinv
        pg = jnp.dot(us[:, :HH], w2g_ref[...],
                     preferred_element_type=jnp.float32) + b2g_ref[...] * nz
        pt = jnp.dot(us[:, HH:], w2t_ref[...],
                     preferred_element_type=jnp.float32) + b2t_ref[...] * nz
        diff = pt - pg + 1e-6
        dist = jnp.sqrt(jnp.sum(diff * diff, axis=1, keepdims=True))
        out_ref[...] = jnp.sum(dist).reshape(1, 1) / G


_final_call = pl.pallas_call(
    _final_body,
    grid=(NPAD // NB,),
    in_specs=[
        pl.BlockSpec((NB, H), lambda i: (i, 0)),
        pl.BlockSpec((NB, H), lambda i: (i, 0)),
        pl.BlockSpec((NB, H), lambda i: (i, 0)),
        pl.BlockSpec((NB, 1), lambda i: (i, 0)),
        pl.BlockSpec((1, NB), lambda i: (0, i)),
        pl.BlockSpec((1, NB), lambda i: (0, i)),
        pl.BlockSpec((G, NB), lambda i: (0, i)),
        pl.BlockSpec((G, NB), lambda i: (0, i)),
        pl.BlockSpec((1, H), lambda i: (0, 0)),
        pl.BlockSpec((HH, OUT), lambda i: (0, 0)),
        pl.BlockSpec((HH, OUT), lambda i: (0, 0)),
        pl.BlockSpec((1, OUT), lambda i: (0, 0)),
        pl.BlockSpec((1, OUT), lambda i: (0, 0)),
    ],
    out_specs=pl.BlockSpec((1, 1), lambda i: (0, 0)),
    out_shape=jax.ShapeDtypeStruct((1, 1), jnp.float32),
    scratch_shapes=[
        pltpu.VMEM((G, H), jnp.float32),
        pltpu.VMEM((G, 1), jnp.float32),
    ],
)


def kernel(x, edge_index, batch, W1g, b1g, W2g, b2g, W1t, b1t, W2t, b2t):
    pad_edges = N + jnp.arange(EPAD - E, dtype=jnp.int32) % (NPAD - N)
    src1 = jnp.concatenate([edge_index[0], pad_edges])
    dst1 = jnp.concatenate([edge_index[1], pad_edges])
    src2 = src1.reshape(NCHUNK, CH)
    dst2 = dst1.reshape(NCHUNK, CH)
    Wcat = jnp.concatenate([W1g, W1t], axis=1)
    b1cat = jnp.concatenate([b1g, b1t]).reshape(1, H)
    ones_ch = jnp.ones((CH,), jnp.float32)
    z1 = jnp.zeros((RPT,), jnp.float32)
    z2 = jnp.zeros((RPT, H), jnp.float32)
    zv = jnp.zeros((VPT,), jnp.float32)
    x_pad = jnp.pad(x, ((0, NPAD - N), (0, 0)))
    batch_pad = jnp.pad(batch, (0, NPAD - N), constant_values=G)

    deg_parts = _deg_call(dst2, ones_ch, z1)
    d0 = deg_parts[0].reshape(NPAD, 1)
    d1 = deg_parts[1].reshape(NPAD, 1)
    h2, dinv = _mm_call(x_pad, Wcat, d0, d1)
    aggp = _agg_call(src1, dst1, h2, z2)
    vp = _vtab_call(src2, dst2, dinv.reshape(NPAD), batch_pad, zv)
    out = _final_call(
        aggp[0], aggp[1], h2, dinv, dinv.reshape(1, NPAD),
        batch_pad.reshape(1, NPAD), vp[0].reshape(G, NPAD),
        vp[1].reshape(G, NPAD), b1cat, W2g, W2t,
        b2g.reshape(1, OUT), b2t.reshape(1, OUT))
    return out.reshape(())


# speedup vs baseline: 1.1108x; 1.0101x over previous
import dataclasses
import functools

import jax
import jax.numpy as jnp
from jax import lax
from jax.experimental import pallas as pl
from jax.experimental.pallas import tpu as pltpu
from jax.experimental.pallas import tpu_sc as plsc

N = 10000
E = 320000
G = 64
CIN = 128
H = 128
HH = 64
OUT = 32
NC, NS = 2, 16
NW = NC * NS
CH = 128
NCHUNK = 2560
EPAD = NCHUNK * CH
C16 = NCHUNK // 16
C32 = NCHUNK // 32
NPAD = 10240
RPT = NPAD // NS
VSIZE = G * NPAD
VPT = VSIZE // NS
NB = 1024

_mesh = plsc.VectorSubcoreMesh(core_axis_name="core", subcore_axis_name="subcore")

_sc_params = pltpu.CompilerParams()
if "needs_layout_passes" in pltpu.CompilerParams.__dataclass_fields__:
    _sc_params = dataclasses.replace(_sc_params, needs_layout_passes=False)


@functools.partial(
    pl.kernel,
    out_type=jax.ShapeDtypeStruct((NC, NPAD), jnp.float32),
    mesh=_mesh,
    scratch_types=[
        pltpu.VMEM((C32, CH), jnp.int32),
        pltpu.VMEM((CH,), jnp.float32),
        pltpu.VMEM_SHARED((NPAD,), jnp.float32),
        pltpu.SemaphoreType.DMA,
    ],
    compiler_params=_sc_params,
)
def _deg_call(dst2_hbm, ones_hbm, z_hbm, deg_out, dstx, onesv, deg_sh, sem):
    c = lax.axis_index("core")
    s = lax.axis_index("subcore")
    w = c * NS + s
    off = w * C32
    pltpu.async_copy(z_hbm, deg_sh.at[pl.ds(s * RPT, RPT)], sem)
    pltpu.async_copy(ones_hbm, onesv, sem)
    pltpu.async_copy(dst2_hbm.at[pl.ds(off, C32)], dstx, sem)
    pltpu.make_async_copy(z_hbm, deg_sh.at[pl.ds(s * RPT, RPT)], sem).wait()
    pltpu.make_async_copy(ones_hbm, onesv, sem).wait()
    pltpu.make_async_copy(dst2_hbm.at[pl.ds(off, C32)], dstx, sem).wait()
    plsc.subcore_barrier()

    @pl.loop(0, C32)
    def _(j):
        pltpu.async_copy(onesv, deg_sh.at[dstx.at[j]], sem, add=True)

    @pl.loop(0, C32)
    def _(j):
        pltpu.make_async_copy(onesv, deg_sh.at[dstx.at[0]], sem).wait()

    plsc.subcore_barrier()
    pltpu.sync_copy(deg_sh.at[pl.ds(s * RPT, RPT)],
                    deg_out.at[c, pl.ds(s * RPT, RPT)])


def _mm_body(x_ref, w_ref, d0_ref, d1_ref, h2_ref, dinv_ref):
    dinv = lax.rsqrt(d0_ref[...] + d1_ref[...] + 1.0)
    h = jnp.dot(x_ref[...], w_ref[...], preferred_element_type=jnp.float32)
    h2_ref[...] = dinv * h
    dinv_ref[...] = dinv


_mm_call = pl.pallas_call(
    _mm_body,
    grid=(NPAD // NB,),
    in_specs=[
        pl.BlockSpec((NB, CIN), lambda i: (i, 0)),
        pl.BlockSpec((CIN, H), lambda i: (0, 0)),
        pl.BlockSpec((NB, 1), lambda i: (i, 0)),
        pl.BlockSpec((NB, 1), lambda i: (i, 0)),
    ],
    out_specs=[
        pl.BlockSpec((NB, H), lambda i: (i, 0)),
        pl.BlockSpec((NB, 1), lambda i: (i, 0)),
    ],
    out_shape=[
        jax.ShapeDtypeStruct((NPAD, H), jnp.float32),
        jax.ShapeDtypeStruct((NPAD, 1), jnp.float32),
    ],
)


CH1 = 80
K1 = (EPAD // NW) // CH1

@functools.partial(
    pl.kernel,
    out_type=jax.ShapeDtypeStruct((NC, NPAD, H), jnp.float32),
    mesh=_mesh,
    scratch_types=(
        [pltpu.VMEM((CH1,), jnp.int32)] * 16
        + [pltpu.VMEM((CH1, H), jnp.float32)] * 4
        + [pltpu.VMEM_SHARED((NPAD, H), jnp.float32)]
        + [pltpu.SemaphoreType.DMA] * 16
    ),
    compiler_params=_sc_params,
)
def _agg_call(src1_hbm, dst1_hbm, h2_hbm, z2_hbm, agg_out,
              sx0, sx1, sx2, sx3, sx4, sx5, sx6, sx7,
              dx0, dx1, dx2, dx3, dx4, dx5, dx6, dx7,
              r0, r1, r2, r3, agg_sh,
              si0, si1, si2, si3, si4, si5, si6, si7,
              sg0, sg1, sg2, sg3, sc0, sc1, sc2, sc3):
    c = lax.axis_index("core")
    s = lax.axis_index("subcore")
    w = c * NS + s
    base = w * K1 * CH1
    sx = (sx0, sx1, sx2, sx3, sx4, sx5, sx6, sx7)
    dx = (dx0, dx1, dx2, dx3, dx4, dx5, dx6, dx7)
    rows = (r0, r1, r2, r3)
    si = (si0, si1, si2, si3, si4, si5, si6, si7)
    sg = (sg0, sg1, sg2, sg3)
    sc = (sc0, sc1, sc2, sc3)

    def fire_idx(j, q):
        e = pl.multiple_of(base + j * CH1, 8)
        pltpu.async_copy(src1_hbm.at[pl.ds(e, CH1)], sx[q], si[q])
        pltpu.async_copy(dst1_hbm.at[pl.ds(e, CH1)], dx[q], si[q])

    def wait_idx(q):
        pltpu.make_async_copy(src1_hbm.at[pl.ds(0, CH1)], sx[q], si[q]).wait()
        pltpu.make_async_copy(dst1_hbm.at[pl.ds(0, CH1)], dx[q], si[q]).wait()

    def fire_gather(q, b):
        pltpu.async_copy(h2_hbm.at[sx[q]], rows[b], sg[b])

    def wait_gather(b):
        pltpu.make_async_copy(h2_hbm.at[sx[0]], rows[b], sg[b]).wait()

    def fire_scatter(q, b):
        pltpu.async_copy(rows[b], agg_sh.at[dx[q]], sc[b], add=True)

    def wait_scatter(b):
        pltpu.make_async_copy(rows[b], agg_sh.at[dx[0]], sc[b]).wait()

    pltpu.async_copy(z2_hbm, agg_sh.at[pl.ds(s * RPT, RPT)], sc0)
    for q in range(6):
        fire_idx(q, q)
    pltpu.make_async_copy(z2_hbm, agg_sh.at[pl.ds(s * RPT, RPT)], sc0).wait()
    plsc.subcore_barrier()

    wait_idx(0)
    fire_gather(0, 0)
    wait_idx(1)
    fire_gather(1, 1)

    @pl.loop(0, K1 // 8)
    def _(i):
        for u in range(8):
            b = u % 4
            b2 = (u + 2) % 4
            q = u
            q2 = (u + 2) % 8
            q6 = (u + 6) % 8
            wait_gather(b)
            fire_scatter(q, b)
            gate2 = pl.when(i >= 1) if u < 2 else (lambda f: f())
            gate_hi = (lambda f: f()) if u < 6 else pl.when(i < K1 // 8 - 1)
            gate_hi2 = (lambda f: f()) if u < 2 else pl.when(i < K1 // 8 - 1)

            @gate2
            def _():
                wait_scatter(b2)

            @gate_hi
            def _():
                wait_idx(q2)
                fire_gather(q2, b2)

            @gate_hi2
            def _():
                fire_idx(8 * i + u + 6, q6)

    wait_scatter(2)
    wait_scatter(3)
    plsc.subcore_barrier()
    pltpu.sync_copy(agg_sh.at[pl.ds(s * RPT, RPT)],
                    agg_out.at[c, pl.ds(s * RPT, RPT)])


@functools.partial(
    pl.kernel,
    out_type=jax.ShapeDtypeStruct((NC, VSIZE), jnp.float32),
    mesh=_mesh,
    scratch_types=[
        pltpu.VMEM((C32, CH), jnp.int32),
        pltpu.VMEM((C32, CH), jnp.int32),
        pltpu.VMEM((C32, CH), jnp.float32),
        pltpu.VMEM((C32, CH), jnp.int32),
        pltpu.VMEM((NPAD,), jnp.float32),
        pltpu.VMEM((NPAD,), jnp.int32),
        pltpu.VMEM_SHARED((VSIZE,), jnp.float32),
        pltpu.SemaphoreType.DMA,
    ],
    compiler_params=_sc_params,
)
def _vtab_call(src2_hbm, dst2_hbm, dinv_hbm, batch_hbm, z1_hbm, v_out,
               srcx, dstx, vvals, vidx, dinvv, batchv, v_sh, sem):
    c = lax.axis_index("core")
    s = lax.axis_index("subcore")
    w = c * NS + s
    off = w * C32
    pltpu.async_copy(z1_hbm, v_sh.at[pl.ds(s * VPT, VPT)], sem)
    pltpu.async_copy(dinv_hbm, dinvv, sem)
    pltpu.async_copy(batch_hbm, batchv, sem)
    pltpu.async_copy(src2_hbm.at[pl.ds(off, C32)], srcx, sem)
    pltpu.async_copy(dst2_hbm.at[pl.ds(off, C32)], dstx, sem)
    pltpu.make_async_copy(z1_hbm, v_sh.at[pl.ds(s * VPT, VPT)], sem).wait()
    pltpu.make_async_copy(dinv_hbm, dinvv, sem).wait()
    pltpu.make_async_copy(batch_hbm, batchv, sem).wait()
    pltpu.make_async_copy(src2_hbm.at[pl.ds(off, C32)], srcx, sem).wait()
    pltpu.make_async_copy(dst2_hbm.at[pl.ds(off, C32)], dstx, sem).wait()

    zf16 = jnp.zeros((16,), jnp.float32)
    zi16 = jnp.zeros((16,), jnp.int32)
    for k in range((NPAD - N) // 16):
        dinvv[pl.ds(N + 16 * k, 16)] = zf16
        batchv[pl.ds(N + 16 * k, 16)] = zi16
    plsc.subcore_barrier()

    @pl.loop(0, C32)
    def _(j):
        @pl.loop(0, CH // 16)
        def _(k):
            s16 = srcx[j, pl.ds(16 * k, 16)]
            d16 = dstx[j, pl.ds(16 * k, 16)]
            vvals[j, pl.ds(16 * k, 16)] = plsc.load_gather(dinvv, [d16])
            vidx[j, pl.ds(16 * k, 16)] = (
                plsc.load_gather(batchv, [d16]) * NPAD + s16)

        pltpu.async_copy(vvals.at[j], v_sh.at[vidx.at[j]], sem, add=True)

    @pl.loop(0, C32)
    def _(j):
        pltpu.make_async_copy(vvals.at[0], v_sh.at[vidx.at[0]], sem).wait()

    plsc.subcore_barrier()
    pltpu.sync_copy(v_sh.at[pl.ds(s * VPT, VPT)],
                    v_out.at[c, pl.ds(s * VPT, VPT)])


def _final_body(a0_ref, a1_ref, h2_ref, dcol_ref, drow_ref, brow_ref, v0_ref,
                v1_ref, b1_ref, w2g_ref, w2t_ref, b2g_ref, b2t_ref, out_ref,
                u_acc, cnt_acc):
    i = pl.program_id(0)

    @pl.when(i == 0)
    def _():
        u_acc[...] = jnp.zeros_like(u_acc)
        cnt_acc[...] = jnp.zeros_like(cnt_acc)

    dinv = dcol_ref[...]
    a = a0_ref[...] + a1_ref[...] + h2_ref[...]
    rd = dinv * jnp.maximum(dinv * a + b1_ref[...], 0.0)
    gids = lax.broadcasted_iota(jnp.int32, (G, NB), 0)
    onehot = brow_ref[...] == gids
    v_eff = v0_ref[...] + v1_ref[...] + jnp.where(onehot, drow_ref[...], 0.0)
    u_acc[...] += jnp.dot(v_eff, rd, preferred_element_type=jnp.float32)
    cnt_acc[...] += jnp.sum(onehot.astype(jnp.float32), axis=1, keepdims=True)

    @pl.when(i == pl.num_programs(0) - 1)
    def _():
        cnt = cnt_acc[...]
        cinv = 1.0 / jnp.maximum(cnt, 1.0)
        nz = jnp.where(cnt > 0, 1.0, 0.0)
        us = u_acc[...] * cinv
        pg = jnp.dot(us[:, :HH], w2g_ref[...],
                     preferred_element_type=jnp.float32) + b2g_ref[...] * nz
        pt = jnp.dot(us[:, HH:], w2t_ref[...],
                     preferred_element_type=jnp.float32) + b2t_ref[...] * nz
        diff = pt - pg + 1e-6
        dist = jnp.sqrt(jnp.sum(diff * diff, axis=1, keepdims=True))
        out_ref[...] = jnp.sum(dist).reshape(1, 1) / G


_final_call = pl.pallas_call(
    _final_body,
    grid=(NPAD // NB,),
    in_specs=[
        pl.BlockSpec((NB, H), lambda i: (i, 0)),
        pl.BlockSpec((NB, H), lambda i: (i, 0)),
        pl.BlockSpec((NB, H), lambda i: (i, 0)),
        pl.BlockSpec((NB, 1), lambda i: (i, 0)),
        pl.BlockSpec((1, NB), lambda i: (0, i)),
        pl.BlockSpec((1, NB), lambda i: (0, i)),
        pl.BlockSpec((G, NB), lambda i: (0, i)),
        pl.BlockSpec((G, NB), lambda i: (0, i)),
        pl.BlockSpec((1, H), lambda i: (0, 0)),
        pl.BlockSpec((HH, OUT), lambda i: (0, 0)),
        pl.BlockSpec((HH, OUT), lambda i: (0, 0)),
        pl.BlockSpec((1, OUT), lambda i: (0, 0)),
        pl.BlockSpec((1, OUT), lambda i: (0, 0)),
    ],
    out_specs=pl.BlockSpec((1, 1), lambda i: (0, 0)),
    out_shape=jax.ShapeDtypeStruct((1, 1), jnp.float32),
    scratch_shapes=[
        pltpu.VMEM((G, H), jnp.float32),
        pltpu.VMEM((G, 1), jnp.float32),
    ],
)


def kernel(x, edge_index, batch, W1g, b1g, W2g, b2g, W1t, b1t, W2t, b2t):
    pad_edges = N + jnp.arange(EPAD - E, dtype=jnp.int32) % (NPAD - N)
    src1 = jnp.concatenate([edge_index[0], pad_edges])
    dst1 = jnp.concatenate([edge_index[1], pad_edges])
    src2 = src1.reshape(NCHUNK, CH)
    dst2 = dst1.reshape(NCHUNK, CH)
    Wcat = jnp.concatenate([W1g, W1t], axis=1)
    b1cat = jnp.concatenate([b1g, b1t]).reshape(1, H)
    ones_ch = jnp.ones((CH,), jnp.float32)
    z1 = jnp.zeros((RPT,), jnp.float32)
    z2 = jnp.zeros((RPT, H), jnp.float32)
    zv = jnp.zeros((VPT,), jnp.float32)
    x_pad = jnp.pad(x, ((0, NPAD - N), (0, 0)))
    batch_pad = jnp.pad(batch, (0, NPAD - N), constant_values=G)

    deg_parts = _deg_call(dst2, ones_ch, z1)
    d0 = deg_parts[0].reshape(NPAD, 1)
    d1 = deg_parts[1].reshape(NPAD, 1)
    h2, dinv = _mm_call(x_pad, Wcat, d0, d1)
    aggp = _agg_call(src1, dst1, h2, z2)
    vp = _vtab_call(src2, dst2, dinv.reshape(NPAD), batch_pad, zv)
    out = _final_call(
        aggp[0], aggp[1], h2, dinv, dinv.reshape(1, NPAD),
        batch_pad.reshape(1, NPAD), vp[0].reshape(G, NPAD),
        vp[1].reshape(G, NPAD), b1cat, W2g, W2t,
        b2g.reshape(1, OUT), b2t.reshape(1, OUT))
    return out.reshape(())
